# baseline (speedup 1.0000x reference)
import functools
import jax
import jax.numpy as jnp
from jax import lax
from jax.experimental import pallas as pl
from jax.experimental.pallas import tpu as pltpu
from jax.experimental.pallas import tpu_sc as plsc

_NUM_NOTES = 100000
_EMBED_DIM = 128
_HIDDEN_DIM = 1024
_NUM_CHORDS = 1000
_TOKENS = 4096 * 20


def _make_sc_gather(total_rows, dim, chunk):
    info = plsc.get_sparse_core_info()
    nw = info.num_cores * info.num_subcores
    rows_per_w = total_rows // nw
    n_chunks = rows_per_w // chunk
    mesh = plsc.VectorSubcoreMesh(core_axis_name="c", subcore_axis_name="s")

    @functools.partial(
        pl.kernel,
        mesh=mesh,
        out_type=jax.ShapeDtypeStruct((total_rows, dim), jnp.float32),
        compiler_params=pltpu.CompilerParams(use_tc_tiling_on_sc=True),
        scratch_types=[
            pltpu.VMEM((chunk,), jnp.int32),
            pltpu.VMEM((chunk, dim), jnp.float32),
            pltpu.SemaphoreType.DMA,
        ],
    )
    def gather_kernel(table_hbm, idx_hbm, out_hbm, idx_v, rows_v, sem):
        wid = lax.axis_index("s") * info.num_cores + lax.axis_index("c")
        base = wid * rows_per_w

        def body(ci, _):
            off = base + ci * chunk
            pltpu.sync_copy(idx_hbm.at[pl.ds(off, chunk)], idx_v)
            pltpu.async_copy(table_hbm.at[idx_v], rows_v, sem).wait()
            pltpu.sync_copy(rows_v, out_hbm.at[pl.ds(off, chunk)])
            return ()

        lax.fori_loop(0, n_chunks, body, (), unroll=False)

    return gather_kernel


def _fuse_weights_kernel(w1_ref, w2_ref, b1_ref, b2_ref, wc_ref, bc_ref):
    wc_ref[...] = jnp.dot(w1_ref[...], w2_ref[...],
                          preferred_element_type=jnp.float32)
    bc_ref[...] = jnp.dot(b1_ref[...], w2_ref[...],
                          preferred_element_type=jnp.float32) + b2_ref[...]


def _fuse_weights(W1, W2, b1, b2):
    return pl.pallas_call(
        _fuse_weights_kernel,
        out_shape=(
            jax.ShapeDtypeStruct((_EMBED_DIM, _NUM_CHORDS), jnp.float32),
            jax.ShapeDtypeStruct((1, _NUM_CHORDS), jnp.float32),
        ),
    )(W1, W2, b1.reshape(1, _HIDDEN_DIM), b2.reshape(1, _NUM_CHORDS))


def _matmul_kernel(x_ref, wc_ref, bc_ref, o_ref):
    o_ref[...] = jnp.dot(x_ref[...], wc_ref[...],
                         preferred_element_type=jnp.float32) + bc_ref[...]


def _matmul(X, Wc, bc, block_m):
    m = X.shape[0]
    grid = (m // block_m,)
    return pl.pallas_call(
        _matmul_kernel,
        grid=grid,
        in_specs=[
            pl.BlockSpec((block_m, _EMBED_DIM), lambda i: (i, 0)),
            pl.BlockSpec((_EMBED_DIM, _NUM_CHORDS), lambda i: (0, 0)),
            pl.BlockSpec((1, _NUM_CHORDS), lambda i: (0, 0)),
        ],
        out_specs=pl.BlockSpec((block_m, _NUM_CHORDS), lambda i: (i, 0)),
        out_shape=jax.ShapeDtypeStruct((m, _NUM_CHORDS), jnp.float32),
        compiler_params=pltpu.CompilerParams(
            dimension_semantics=("arbitrary",),
        ),
    )(X, Wc, bc)


@jax.jit
def kernel(input_notes, emb, W1, b1, W2, b2):
    batch, hist = input_notes.shape
    idx = input_notes.reshape(-1).astype(jnp.int32)
    gather = _make_sc_gather(_TOKENS, _EMBED_DIM, chunk=256)
    X = gather(emb, idx)
    Wc, bc = _fuse_weights(W1, W2, b1, b2)
    out = _matmul(X, Wc, bc, block_m=2048)
    return out.reshape(batch, hist, _NUM_CHORDS)


# speedup vs baseline: 4.2993x; 4.2993x over previous
import functools
import jax
import jax.numpy as jnp
from jax import lax
from jax.experimental import pallas as pl
from jax.experimental.pallas import tpu as pltpu
from jax.experimental.pallas import tpu_sc as plsc

_EMBED_DIM = 128
_HIDDEN_DIM = 1024
_NUM_CHORDS = 1000
_TOKENS = 4096 * 20


def _make_sc_gather(total_rows, dim, chunk):
    info = plsc.get_sparse_core_info()
    nw = info.num_cores * info.num_subcores
    rows_per_w = total_rows // nw
    n_chunks = rows_per_w // chunk
    mesh = plsc.VectorSubcoreMesh(core_axis_name="c", subcore_axis_name="s")

    @functools.partial(
        pl.kernel,
        mesh=mesh,
        out_type=jax.ShapeDtypeStruct((total_rows, dim), jnp.float32),
        scratch_types=[
            pltpu.VMEM((chunk,), jnp.int32),
            pltpu.VMEM((chunk, dim), jnp.float32),
            pltpu.SemaphoreType.DMA,
        ],
    )
    def gather_kernel(table_hbm, idx_hbm, out_hbm, idx_v, rows_v, sem):
        wid = lax.axis_index("s") * info.num_cores + lax.axis_index("c")
        base = wid * rows_per_w

        def body(ci, _):
            off = base + ci * chunk
            pltpu.sync_copy(idx_hbm.at[pl.ds(off, chunk)], idx_v)
            pltpu.async_copy(table_hbm.at[idx_v], rows_v, sem).wait()
            pltpu.sync_copy(rows_v, out_hbm.at[pl.ds(off, chunk)])
            return ()

        lax.fori_loop(0, n_chunks, body, (), unroll=False)

    return gather_kernel


def _fuse_weights_kernel(w1_ref, w2_ref, b1_ref, b2_ref, wct_ref, bcc_ref):
    wct_ref[...] = lax.dot_general(
        w2_ref[...], w1_ref[...], (((0,), (1,)), ((), ())),
        preferred_element_type=jnp.float32)
    bcc_ref[...] = lax.dot_general(
        w2_ref[...], b1_ref[...], (((0,), (1,)), ((), ())),
        preferred_element_type=jnp.float32) + b2_ref[...]


def _fuse_weights(W1, W2, b1, b2):
    return pl.pallas_call(
        _fuse_weights_kernel,
        out_shape=(
            jax.ShapeDtypeStruct((_NUM_CHORDS, _EMBED_DIM), jnp.float32),
            jax.ShapeDtypeStruct((_NUM_CHORDS, 1), jnp.float32),
        ),
    )(W1, W2, b1.reshape(1, _HIDDEN_DIM), b2.reshape(_NUM_CHORDS, 1))


def _matmul_t_kernel(x_ref, wct_ref, bcc_ref, o_ref):
    o_ref[0] = lax.dot_general(
        wct_ref[...], x_ref[0], (((1,), (1,)), ((), ())),
        preferred_element_type=jnp.float32) + bcc_ref[...]


def _matmul_t(X3, WcT, bcc, hist, block_b):
    batch = X3.shape[1]
    grid = (hist, batch // block_b)
    return pl.pallas_call(
        _matmul_t_kernel,
        grid=grid,
        in_specs=[
            pl.BlockSpec((1, block_b, _EMBED_DIM), lambda l, j: (l, j, 0)),
            pl.BlockSpec((_NUM_CHORDS, _EMBED_DIM), lambda l, j: (0, 0)),
            pl.BlockSpec((_NUM_CHORDS, 1), lambda l, j: (0, 0)),
        ],
        out_specs=pl.BlockSpec((1, _NUM_CHORDS, block_b),
                               lambda l, j: (l, 0, j)),
        out_shape=jax.ShapeDtypeStruct((hist, _NUM_CHORDS, batch),
                                       jnp.float32),
        compiler_params=pltpu.CompilerParams(
            dimension_semantics=("arbitrary", "arbitrary"),
        ),
    )(X3, WcT, bcc)


@jax.jit
def kernel(input_notes, emb, W1, b1, W2, b2):
    batch, hist = input_notes.shape
    idx = jnp.transpose(input_notes).reshape(-1).astype(jnp.int32)
    gather = _make_sc_gather(_TOKENS, _EMBED_DIM, chunk=256)
    X = gather(emb, idx)
    X3 = X.reshape(hist, batch, _EMBED_DIM)
    WcT, bcc = _fuse_weights(W1, W2, b1, b2)
    out_t = _matmul_t(X3, WcT, bcc, hist, block_b=2048)
    return jnp.transpose(out_t, (2, 0, 1))
